# baseline (speedup 1.0000x reference)
import dataclasses
import functools

import jax
import jax.numpy as jnp
from jax import lax
from jax.experimental import pallas as pl
from jax.experimental.pallas import tpu as pltpu
from jax.experimental.pallas import tpu_sc as plsc

B, C, H, W = 16, 256, 32, 32
HW = H * W
K = 1024

_SC_INFO = plsc.get_sparse_core_info()
NC, NS, L = _SC_INFO.num_cores, _SC_INFO.num_subcores, _SC_INFO.num_lanes
NW = NC * NS
MB = 4
SB = 8
PPW = SB * HW // NW
GCH = 128
NCHUNKS = PPW // GCH


def _prep_pn(b, bank_ref, pn_t_ref):
    @pl.when(b == 0)
    def _():
        bank_t = lax.transpose(bank_ref[...], (1, 0))
        norm = jnp.sqrt(jnp.sum(bank_t * bank_t, axis=0, keepdims=True))
        pn_t_ref[...] = bank_t / jnp.maximum(norm, 1e-12)


def _similarity_idx(x_ref, pn_t_ref):
    xb = x_ref[...].reshape(MB * HW, C)
    xnorm = jnp.sqrt(jnp.sum(xb * xb, axis=1, keepdims=True))
    xn = xb / jnp.maximum(xnorm, 1e-12)
    sim = jnp.dot(xn, pn_t_ref[...], preferred_element_type=jnp.float32)
    return jnp.argmax(sim, axis=1).astype(jnp.int32)


def _match_kernel(x_ref, bank_ref, idx_ref, pn_t_ref):
    _prep_pn(pl.program_id(0), bank_ref, pn_t_ref)
    idx_col = _similarity_idx(x_ref, pn_t_ref)
    idx_ref[...] = idx_col.reshape(MB, HW // 128, 128)


def _match_gather_kernel(x_ref, bank_ref, idx_ref, rows_ref, pn_t_ref):
    _prep_pn(pl.program_id(0), bank_ref, pn_t_ref)
    idx_col = _similarity_idx(x_ref, pn_t_ref)
    idx_ref[...] = idx_col.reshape(MB, HW // 128, 128)
    iota_k = lax.broadcasted_iota(jnp.int32, (MB * HW, K), 1)
    onehot = (iota_k == idx_col[:, None]).astype(jnp.float32)
    rows_ref[...] = jnp.dot(onehot, bank_ref[...],
                            preferred_element_type=jnp.float32)


def _match(xr, bank, nb, boff):
    return pl.pallas_call(
        _match_kernel,
        grid=(nb // MB,),
        in_specs=[
            pl.BlockSpec((MB, HW, C), lambda b: (b + boff // MB, 0, 0)),
            pl.BlockSpec((K, C), lambda b: (0, 0)),
        ],
        out_specs=pl.BlockSpec((MB, HW // 128, 128), lambda b: (b, 0, 0)),
        out_shape=jax.ShapeDtypeStruct((nb, HW // 128, 128), jnp.int32),
        scratch_shapes=[pltpu.VMEM((C, K), jnp.float32)],
    )(xr, bank)


def _match_gather(xr, bank, nb, boff):
    return pl.pallas_call(
        _match_gather_kernel,
        grid=(nb // MB,),
        in_specs=[
            pl.BlockSpec((MB, HW, C), lambda b: (b + boff // MB, 0, 0)),
            pl.BlockSpec((K, C), lambda b: (0, 0)),
        ],
        out_specs=[
            pl.BlockSpec((MB, HW // 128, 128), lambda b: (b, 0, 0)),
            pl.BlockSpec((MB * HW, C), lambda b: (b, 0)),
        ],
        out_shape=[
            jax.ShapeDtypeStruct((nb, HW // 128, 128), jnp.int32),
            jax.ShapeDtypeStruct((nb * HW, C), jnp.float32),
        ],
        scratch_shapes=[pltpu.VMEM((C, K), jnp.float32)],
    )(xr, bank)


_SC_PARAMS = pltpu.CompilerParams()
if "needs_layout_passes" in pltpu.CompilerParams.__dataclass_fields__:
    _SC_PARAMS = dataclasses.replace(_SC_PARAMS, needs_layout_passes=False)


@functools.partial(
    pl.kernel,
    mesh=plsc.VectorSubcoreMesh(core_axis_name="c", subcore_axis_name="s"),
    compiler_params=_SC_PARAMS,
    out_type=jax.ShapeDtypeStruct((SB * HW, C), jnp.float32),
    scratch_types=[
        pltpu.VMEM((NCHUNKS, GCH), jnp.int32),
        pltpu.VMEM((3, GCH, C), jnp.float32),
        pltpu.SemaphoreType.DMA,
        pltpu.SemaphoreType.DMA,
    ],
)
def _sc_gather(bank_hbm, idx_hbm, out_hbm, idxv, rows, gsem, wsem):
    wid = lax.axis_index("s") * NC + lax.axis_index("c")
    base = wid * PPW
    pltpu.sync_copy(idx_hbm.at[pl.ds(wid * NCHUNKS, NCHUNKS)], idxv)

    def _g(c):
        return pltpu.make_async_copy(
            bank_hbm.at[idxv.at[c]], rows.at[c % 3], gsem)

    def _w(c):
        return pltpu.make_async_copy(
            rows.at[c % 3], out_hbm.at[pl.ds(base + c * GCH, GCH)], wsem)

    _g(0).start()
    _g(1).start()
    for c in range(NCHUNKS):
        _g(c).wait()
        _w(c).start()
        if c >= 1:
            _w(c - 1).wait()
        if c + 2 < NCHUNKS:
            _g(c + 2).start()
    _w(NCHUNKS - 1).wait()


def kernel(x, prototype_bank):
    xr = x.transpose(0, 2, 3, 1).reshape(B, HW, C)
    idx_a8 = _match(xr, prototype_bank, SB, 0)
    idx_a = idx_a8.reshape(SB, HW)
    rows_sc = _sc_gather(
        prototype_bank, idx_a.reshape(SB * HW // GCH, GCH))
    idx_b8, rows_tc = _match_gather(xr, prototype_bank, B - SB, SB)
    idx = jnp.concatenate([idx_a, idx_b8.reshape(B - SB, HW)], axis=0)
    rows = jnp.concatenate([rows_sc, rows_tc], axis=0)
    out = rows.reshape(B, H, W, C).transpose(0, 3, 1, 2)
    return out, idx


# speedup vs baseline: 1.2368x; 1.2368x over previous
import dataclasses
import functools

import jax
import jax.numpy as jnp
from jax import lax
from jax.experimental import pallas as pl
from jax.experimental.pallas import tpu as pltpu
from jax.experimental.pallas import tpu_sc as plsc

B, C, H, W = 16, 256, 32, 32
HW = H * W
K = 1024

_SC_INFO = plsc.get_sparse_core_info()
NC, NS, L = _SC_INFO.num_cores, _SC_INFO.num_subcores, _SC_INFO.num_lanes
NW = NC * NS
MB = 4
PPW = B * HW // NW
GCH = 128
NCHUNKS = PPW // GCH


def _match_kernel(x_ref, bank_ref, idx_ref, pn_t_ref):
    b = pl.program_id(0)

    @pl.when(b == 0)
    def _():
        bank_t = lax.transpose(bank_ref[...], (1, 0))
        norm = jnp.sqrt(jnp.sum(bank_t * bank_t, axis=0, keepdims=True))
        pn_t_ref[...] = bank_t / jnp.maximum(norm, 1e-12)

    xb = x_ref[...].reshape(MB * HW, C)
    xnorm = jnp.sqrt(jnp.sum(xb * xb, axis=1, keepdims=True))
    xn = xb / jnp.maximum(xnorm, 1e-12)

    sim = jnp.dot(xn, pn_t_ref[...], preferred_element_type=jnp.float32)

    idx_col = jnp.argmax(sim, axis=1).astype(jnp.int32)
    idx_ref[...] = idx_col.reshape(MB, HW // 128, 128)


def _match(xr, bank):
    return pl.pallas_call(
        _match_kernel,
        grid=(B // MB,),
        in_specs=[
            pl.BlockSpec((MB, HW, C), lambda b: (b, 0, 0)),
            pl.BlockSpec((K, C), lambda b: (0, 0)),
        ],
        out_specs=pl.BlockSpec((MB, HW // 128, 128), lambda b: (b, 0, 0)),
        out_shape=jax.ShapeDtypeStruct((B, HW // 128, 128), jnp.int32),
        scratch_shapes=[pltpu.VMEM((C, K), jnp.float32)],
    )(xr, bank)


_SC_PARAMS = pltpu.CompilerParams()
if "needs_layout_passes" in pltpu.CompilerParams.__dataclass_fields__:
    _SC_PARAMS = dataclasses.replace(_SC_PARAMS, needs_layout_passes=False)


@functools.partial(
    pl.kernel,
    mesh=plsc.VectorSubcoreMesh(core_axis_name="c", subcore_axis_name="s"),
    compiler_params=_SC_PARAMS,
    out_type=jax.ShapeDtypeStruct((B * HW, C), jnp.float32),
    scratch_types=[
        pltpu.VMEM((NCHUNKS, GCH), jnp.int32),
        pltpu.VMEM((3, GCH, C), jnp.float32),
        pltpu.SemaphoreType.DMA,
        pltpu.SemaphoreType.DMA,
    ],
)
def _sc_gather(bank_hbm, idx_hbm, out_hbm, idxv, rows, gsem, wsem):
    wid = lax.axis_index("s") * NC + lax.axis_index("c")
    base = wid * PPW
    pltpu.sync_copy(idx_hbm.at[pl.ds(wid * NCHUNKS, NCHUNKS)], idxv)

    def _g(c):
        return pltpu.make_async_copy(
            bank_hbm.at[idxv.at[c]], rows.at[c % 3], gsem)

    def _w(c):
        return pltpu.make_async_copy(
            rows.at[c % 3], out_hbm.at[pl.ds(base + c * GCH, GCH)], wsem)

    _g(0).start()
    _g(1).start()
    for c in range(NCHUNKS):
        _g(c).wait()
        _w(c).start()
        if c >= 1:
            _w(c - 1).wait()
        if c + 2 < NCHUNKS:
            _g(c + 2).start()
    _w(NCHUNKS - 1).wait()


def kernel(x, prototype_bank):
    xr = x.transpose(0, 2, 3, 1).reshape(B, HW, C)
    idx8 = _match(xr, prototype_bank)
    idx = idx8.reshape(B, HW)
    idx2 = idx.reshape(B * HW // GCH, GCH)
    rows = _sc_gather(prototype_bank, idx2)
    out = rows.reshape(B, H, W, C).transpose(0, 3, 1, 2)
    return out, idx
